# Initial kernel scaffold; baseline (speedup 1.0000x reference)
import jax
import jax.numpy as jnp
from jax.experimental import pallas as pl


def kernel(features, edge_index, edge_weight, W1, O1, W2, O2, W3, O3, W4, O4, W5, O5, V0w, V0b, V1w, V1b, V2w, V2b, V3w, V3b, Vw, Vb):
    raise NotImplementedError("write your pallas kernel here")


# speedup vs baseline: 15.1796x; 15.1796x over previous
import functools

import numpy as np
import jax
import jax.numpy as jnp
from jax import lax
from jax.experimental import pallas as pl
from jax.experimental.pallas import tpu as pltpu
from jax.experimental.pallas import tpu_sc as plsc

KAPPA = 0.9
NC, NS, LANES = 2, 16, 16
NW = NC * NS
CE = 128
CH = 79
GP = CH * CE
NITER = 50
NP = 10240
BLK = 512

_GDN = lax.GatherDimensionNumbers(
    offset_dims=(), collapsed_slice_dims=(0,), start_index_map=(0,))


def _splat(vec, lane):
    idx = jnp.full((LANES, 1), lane, jnp.int32)
    return lax.gather(vec, idx, _GDN, slice_sizes=(1,),
                      mode=lax.GatherScatterMode.PROMISE_IN_BOUNDS)


def _rsq(x):
    xs = jnp.maximum(x, np.float32(1e-30))
    i = lax.bitcast_convert_type(xs, jnp.int32)
    i = np.int32(0x5F3759DF) - lax.shift_right_logical(i, 1)
    y = lax.bitcast_convert_type(i, jnp.float32)
    for _ in range(4):
        y = y * (np.float32(1.5) - np.float32(0.5) * xs * y * y)
    return y


@functools.cache
def _make_spmm(n, m):
    mesh = plsc.VectorSubcoreMesh(core_axis_name="c", subcore_axis_name="s")
    rpt = n // NS
    grp = m // LANES

    def body(y_hbm, row_hbm, col_hbm, w_hbm, out_hbm,
             rowt, colt, wt, gbuf, sem, accum):
        c = lax.axis_index("c")
        s = lax.axis_index("s")
        wid = c * NS + s
        pltpu.sync_copy(row_hbm.at[wid], rowt)
        pltpu.sync_copy(col_hbm.at[wid], colt)
        pltpu.sync_copy(w_hbm.at[wid], wt)

        def _z(i, _):
            for j in range(grp):
                gbuf[i, pl.ds(j * LANES, LANES)] = jnp.zeros((LANES,), jnp.float32)
            return 0
        lax.fori_loop(0, CE, _z, 0)
        base = s * rpt
        for q in range(rpt // CE):
            pltpu.sync_copy(gbuf, accum.at[pl.ds(base + q * CE, CE)])
        plsc.subcore_barrier()

        def chunk(k, _):
            pltpu.async_copy(y_hbm.at[rowt.at[k]], gbuf, sem).wait()

            def grp16(g, _):
                wv = wt[k, pl.ds(g * LANES, LANES)]
                for lane in range(LANES):
                    ws = _splat(wv, lane)
                    e = g * LANES + lane
                    for j in range(grp):
                        gbuf[e, pl.ds(j * LANES, LANES)] = (
                            gbuf[e, pl.ds(j * LANES, LANES)] * ws)
                return 0
            lax.fori_loop(0, CE // LANES, grp16, 0)
            pltpu.sync_copy(gbuf, accum.at[colt.at[k]], add=True)
            return 0
        lax.fori_loop(0, CH, chunk, 0)
        plsc.subcore_barrier()
        pltpu.sync_copy(accum.at[pl.ds(base, rpt)],
                        out_hbm.at[c, pl.ds(base, rpt)])

    return pl.kernel(
        body,
        out_type=jax.ShapeDtypeStruct((NC, n, m), jnp.float32),
        mesh=mesh,
        compiler_params=pltpu.CompilerParams(needs_layout_passes=False, use_tc_tiling_on_sc=False),
        scratch_types=[
            pltpu.VMEM((CH, CE), jnp.int32),
            pltpu.VMEM((CH, CE), jnp.int32),
            pltpu.VMEM((CH, CE), jnp.float32),
            pltpu.VMEM((CE, m), jnp.float32),
            pltpu.SemaphoreType.DMA,
            pltpu.VMEM_SHARED((n, m), jnp.float32),
        ],
    )


@functools.cache
def _make_rho(n):
    mesh = plsc.VectorSubcoreMesh(core_axis_name="c", subcore_axis_name="s")
    npad = ((n + NS * LANES - 1) // (NS * LANES)) * NS * LANES
    spt = npad // NS

    def body(row_hbm, col_hbm, w_hbm, out_hbm,
             rowt, colt, wt, pb, vt, avt, zb, sb, pbt, sem, accum, parts):
        c = lax.axis_index("c")
        s = lax.axis_index("s")
        for h in range(2):
            pltpu.sync_copy(row_hbm.at[s + h * NS], rowt.at[pl.ds(h * CH, CH)])
            pltpu.sync_copy(col_hbm.at[s + h * NS], colt.at[pl.ds(h * CH, CH)])
            pltpu.sync_copy(w_hbm.at[s + h * NS], wt.at[pl.ds(h * CH, CH)])

        v0 = jnp.full((LANES,), np.float32(1.0) / np.float32(np.sqrt(n)),
                      jnp.float32)
        zv = jnp.zeros((LANES,), jnp.float32)

        def _iv(i, _):
            vt[pl.ds(i * LANES, LANES)] = v0
            return 0
        lax.fori_loop(0, n // LANES, _iv, 0)

        def _ivz(i, _):
            vt[pl.ds(i * LANES, LANES)] = zv
            return 0
        lax.fori_loop(n // LANES, npad // LANES, _ivz, 0)
        for i in range(spt // LANES):
            zb[pl.ds(i * LANES, LANES)] = zv

        def it(t, s2_prev):
            pltpu.sync_copy(zb, accum.at[pl.ds(s * spt, spt)])
            plsc.subcore_barrier()

            def prow(r, _):
                for j in range(CE // LANES):
                    cv = colt[r, pl.ds(j * LANES, LANES)]
                    wv = wt[r, pl.ds(j * LANES, LANES)]
                    vv = plsc.load_gather(vt, [cv])
                    pb[r, pl.ds(j * LANES, LANES)] = vv * wv
                return 0
            lax.fori_loop(0, 2 * CH, prow, 0)
            descs = [pltpu.async_copy(pb.at[r], accum.at[rowt.at[r]], sem,
                                      add=True)
                     for r in range(2 * CH)]
            for d in descs:
                d.wait()
            plsc.subcore_barrier()

            pltpu.sync_copy(accum, avt)

            def sq(i, acc):
                x = avt[pl.ds(s * spt + i * LANES, LANES)]
                return acc + x * x
            part = lax.fori_loop(0, spt // LANES, sq,
                                 jnp.zeros((LANES,), jnp.float32))
            sb[...] = part
            pltpu.sync_copy(sb, parts.at[s])
            plsc.subcore_barrier()
            pltpu.sync_copy(parts, pbt)
            tot = jnp.zeros((LANES,), jnp.float32)
            for t2 in range(NS):
                tot = tot + pbt[t2]
            s2 = jnp.sum(tot)
            inv = _rsq(s2)

            def up(i, _):
                vt[pl.ds(i * LANES, LANES)] = avt[pl.ds(i * LANES, LANES)] * inv
                return 0
            lax.fori_loop(0, npad // LANES, up, 0)
            plsc.subcore_barrier()
            return s2
        s2f = lax.fori_loop(0, NITER, it, jnp.float32(0.0))
        rho = s2f * _rsq(s2f)

        @pl.when((c == 0) & (s == 0))
        def _():
            sb[...] = jnp.full((LANES,), np.float32(0.0)) + rho
            pltpu.sync_copy(sb, out_hbm)

    return pl.kernel(
        body,
        out_type=jax.ShapeDtypeStruct((LANES,), jnp.float32),
        mesh=mesh,
        compiler_params=pltpu.CompilerParams(needs_layout_passes=False, use_tc_tiling_on_sc=False),
        scratch_types=[
            pltpu.VMEM((2 * CH, CE), jnp.int32),
            pltpu.VMEM((2 * CH, CE), jnp.int32),
            pltpu.VMEM((2 * CH, CE), jnp.float32),
            pltpu.VMEM((2 * CH, CE), jnp.float32),
            pltpu.VMEM((npad,), jnp.float32),
            pltpu.VMEM((npad,), jnp.float32),
            pltpu.VMEM((spt,), jnp.float32),
            pltpu.VMEM((LANES,), jnp.float32),
            pltpu.VMEM((NS, LANES), jnp.float32),
            pltpu.SemaphoreType.DMA,
            pltpu.VMEM_SHARED((npad,), jnp.float32),
            pltpu.VMEM_SHARED((NS, LANES), jnp.float32),
        ],
    )


@functools.cache
def _mm(n, kin, kout):
    def body(z_ref, w_ref, o_ref):
        o_ref[...] = jnp.dot(z_ref[...], w_ref[...],
                             preferred_element_type=jnp.float32)
    return pl.pallas_call(
        body,
        grid=(n // BLK,),
        in_specs=[pl.BlockSpec((BLK, kin), lambda i: (i, 0)),
                  pl.BlockSpec((kin, kout), lambda i: (0, 0))],
        out_specs=pl.BlockSpec((BLK, kout), lambda i: (i, 0)),
        out_shape=jax.ShapeDtypeStruct((n, kout), jnp.float32),
    )


@functools.cache
def _relu_mm(n, m, with_p):
    def body(*refs):
        if with_p:
            p_ref, b_ref, w_ref, o_ref = refs
            x = p_ref[0] + p_ref[1] + b_ref[0] + b_ref[1]
        else:
            b_ref, w_ref, o_ref = refs
            x = b_ref[0] + b_ref[1]
        o_ref[...] = jnp.dot(jnp.maximum(x, 0.0), w_ref[...],
                             preferred_element_type=jnp.float32)
    pspec = pl.BlockSpec((NC, BLK, m), lambda i: (0, i, 0))
    in_specs = ([pspec, pspec] if with_p else [pspec]) + [
        pl.BlockSpec((m, m), lambda i: (0, 0))]
    return pl.pallas_call(
        body,
        grid=(n // BLK,),
        in_specs=in_specs,
        out_specs=pl.BlockSpec((BLK, m), lambda i: (i, 0)),
        out_shape=jax.ShapeDtypeStruct((n, m), jnp.float32),
    )


@functools.cache
def _comb(n, m, kin, act):
    def body(p_ref, b_ref, z_ref, w_ref, bias_ref, o_ref):
        x = jnp.maximum(p_ref[0] + p_ref[1] + b_ref[0] + b_ref[1], 0.0)
        y = x + jnp.dot(z_ref[...], w_ref[...],
                        preferred_element_type=jnp.float32) + bias_ref[...]
        o_ref[...] = jnp.where(y > 0, y, jnp.exp(y) - 1.0) if act else y
    pspec = pl.BlockSpec((NC, BLK, m), lambda i: (0, i, 0))
    return pl.pallas_call(
        body,
        grid=(n // BLK,),
        in_specs=[pspec, pspec,
                  pl.BlockSpec((BLK, kin), lambda i: (i, 0)),
                  pl.BlockSpec((kin, m), lambda i: (0, 0)),
                  pl.BlockSpec((1, m), lambda i: (0, 0))],
        out_specs=pl.BlockSpec((BLK, m), lambda i: (i, 0)),
        out_shape=jax.ShapeDtypeStruct((n, m), jnp.float32),
    )


def _proj(W, v):
    a = jnp.abs(W)
    asort = jnp.sort(a, axis=1)[:, ::-1]
    cssv = jnp.cumsum(asort, axis=1) - v
    ind = jnp.arange(1, W.shape[1] + 1, dtype=W.dtype)
    cond = (asort - cssv / ind) > 0
    rho = jnp.maximum(jnp.sum(cond, axis=1), 1)
    theta = cssv[jnp.arange(W.shape[0]), rho - 1] / rho.astype(W.dtype)
    proj = jnp.sign(W) * jnp.maximum(a - theta[:, None], 0.0)
    return jnp.where((jnp.sum(a, axis=1) > v)[:, None], proj, W)


def kernel(features, edge_index, edge_weight, W1, O1, W2, O2, W3, O3, W4, O4,
           W5, O5, V0w, V0b, V1w, V1b, V2w, V2b, V3w, V3b, Vw, Vb):
    n = features.shape[1]
    e = edge_weight.shape[0]
    epg = e // NW
    pad = GP - epg
    row = edge_index[0].reshape(NW, epg)
    col = edge_index[1].reshape(NW, epg)
    rowg = jnp.pad(row, ((0, 0), (0, pad))).reshape(NW, CH, CE)
    colg = jnp.pad(col, ((0, 0), (0, pad))).reshape(NW, CH, CE)
    wg = jnp.pad(edge_weight.reshape(NW, epg),
                 ((0, 0), (0, pad))).reshape(NW, CH, CE)

    a_rho = _make_rho(n)(rowg, colg, wg)[0]
    radius = KAPPA / a_rho

    z = jnp.pad(features.T, ((0, NP - n), (0, 0)))
    layers = [(W1, O1, V0w, V0b), (W2, O2, V1w, V1b), (W3, O3, V2w, V2b),
              (W4, O4, V3w, V3b), (W5, O5, Vw, Vb)]
    for li, (W, O, Vw_, Vb_) in enumerate(layers):
        m, p = O.shape
        Wp = _proj(W, radius)
        spmm = _make_spmm(NP, m)
        s = _mm(NP, p, m)(z, O.T)
        b_parts = spmm(s, rowg, colg, wg)
        h = _relu_mm(NP, m, False)(b_parts, Wp.T)
        for _ in range(8):
            p_parts = spmm(h, rowg, colg, wg)
            h = _relu_mm(NP, m, True)(p_parts, b_parts, Wp.T)
        p_parts = spmm(h, rowg, colg, wg)
        z = _comb(NP, m, p, li < 4)(p_parts, b_parts, z, Vw_.T,
                                    Vb_.reshape(1, m))
    return z[:n]
